# baseline (speedup 1.0000x reference)
import functools

import jax
import jax.numpy as jnp
from jax import lax
from jax.experimental import pallas as pl
from jax.experimental.pallas import tpu as pltpu
from jax.experimental.pallas import tpu_sc as plsc

N = 10000
F = 128
E = 320000
NC = 2
NS = 16
NW = NC * NS
EPT = E // NW
WIN = 80
NWIN = EPT // WIN
TR = 624
SR = 208
TAILB = NS * TR
DW = 16

_mesh = plsc.VectorSubcoreMesh(core_axis_name="c", subcore_axis_name="s")


def _al8(i):
    return pl.multiple_of(i, 8)


@functools.partial(
    pl.kernel,
    out_type=jax.ShapeDtypeStruct((NC, N, F), jnp.float32),
    mesh=_mesh,
    scratch_types=[
        pltpu.VMEM_SHARED((N, F), jnp.float32),
        pltpu.VMEM((NWIN, WIN), jnp.int32),
        pltpu.VMEM((WIN, F), jnp.float32),
    ],
)
def _sc_deg(dst_hbm, out_hbm, dacc, dst_all, obuf):
    cid = lax.axis_index("c")
    sid = lax.axis_index("s")
    wid = cid * NS + sid
    rbase = sid * TR

    @pl.loop(0, WIN)
    def _(i):
        @pl.loop(0, F, step=16)
        def _(j):
            obuf[i, pl.ds(j, 16)] = jnp.zeros((16,), jnp.float32)

    @pl.loop(0, 7 * WIN, step=WIN)
    def _(r):
        pltpu.sync_copy(obuf, dacc.at[pl.ds(_al8(rbase + r), WIN)])

    pltpu.sync_copy(obuf.at[pl.ds(0, 64)], dacc.at[pl.ds(_al8(rbase + 7 * WIN), 64)])

    @pl.when(sid == NS - 1)
    def _():
        pltpu.sync_copy(obuf.at[pl.ds(0, 16)], dacc.at[pl.ds(TAILB, 16)])

    @pl.loop(0, WIN)
    def _(i):
        @pl.loop(0, F, step=16)
        def _(j):
            obuf[i, pl.ds(j, 16)] = jnp.ones((16,), jnp.float32)

    pltpu.sync_copy(dst_hbm.at[wid], dst_all)
    plsc.subcore_barrier()

    @pl.loop(0, NWIN)
    def _(w):
        pltpu.sync_copy(obuf, dacc.at[dst_all.at[w]], add=True)

    plsc.subcore_barrier()

    pltpu.sync_copy(dacc.at[pl.ds(_al8(rbase), TR)],
                    out_hbm.at[cid, pl.ds(_al8(rbase), TR)])

    @pl.when(sid == NS - 1)
    def _():
        pltpu.sync_copy(dacc.at[pl.ds(TAILB, 16)], out_hbm.at[cid, pl.ds(TAILB, 16)])


@functools.partial(
    pl.kernel,
    out_type=jax.ShapeDtypeStruct((NC, N, F), jnp.float32),
    mesh=_mesh,
    scratch_types=[
        pltpu.VMEM_SHARED((N, F), jnp.float32),
        pltpu.VMEM((EPT,), jnp.int32),
        pltpu.VMEM((NWIN, WIN), jnp.int32),
        pltpu.VMEM((WIN, F), jnp.float32),
        pltpu.VMEM((WIN, F), jnp.float32),
        pltpu.SemaphoreType.DMA,
        pltpu.SemaphoreType.DMA,
        pltpu.SemaphoreType.DMA,
        pltpu.SemaphoreType.DMA,
    ],
)
def _sc_conv(zs_hbm, src_hbm, dst_hbm, out_hbm,
             acc, src_all, dst_all, row0, row1, gsem0, gsem1, ssem0, ssem1):
    cid = lax.axis_index("c")
    sid = lax.axis_index("s")
    wid = cid * NS + sid
    rbase = sid * TR

    @pl.loop(0, WIN)
    def _(i):
        @pl.loop(0, F, step=16)
        def _(j):
            row0[i, pl.ds(j, 16)] = jnp.zeros((16,), jnp.float32)

    @pl.loop(0, 7 * WIN, step=WIN)
    def _(r):
        pltpu.sync_copy(row0, acc.at[pl.ds(_al8(rbase + r), WIN)])

    pltpu.sync_copy(row0.at[pl.ds(0, 64)], acc.at[pl.ds(_al8(rbase + 7 * WIN), 64)])

    @pl.when(sid == NS - 1)
    def _():
        pltpu.sync_copy(row0.at[pl.ds(0, 16)], acc.at[pl.ds(TAILB, 16)])

    pltpu.sync_copy(src_hbm.at[wid, 0], src_all)
    pltpu.sync_copy(dst_hbm.at[wid], dst_all)
    plsc.subcore_barrier()

    def start_gather(w, row, sem):
        pltpu.async_copy(zs_hbm.at[src_all.at[pl.ds(_al8(w * WIN), WIN)]], row, sem)

    def wait_gather(w, row, sem):
        pltpu.make_async_copy(
            zs_hbm.at[src_all.at[pl.ds(_al8(w * WIN), WIN)]], row, sem).wait()

    def start_scatter(w, row, sem):
        pltpu.async_copy(row, acc.at[dst_all.at[w]], sem, add=True)

    def wait_scatter(w, row, sem):
        pltpu.make_async_copy(row, acc.at[dst_all.at[w]], sem).wait()

    start_gather(jnp.int32(0), row0, gsem0)
    start_gather(jnp.int32(1), row1, gsem1)

    @pl.loop(0, NWIN - 1, step=2)
    def _(k):
        wait_gather(k, row0, gsem0)
        start_scatter(k, row0, ssem0)
        wait_gather(k + 1, row1, gsem1)
        wait_scatter(k, row0, ssem0)
        start_gather(k + 2, row0, gsem0)
        start_scatter(k + 1, row1, ssem1)
        wait_scatter(k + 1, row1, ssem1)

        @pl.when(k + 3 < NWIN)
        def _():
            start_gather(k + 3, row1, gsem1)

    wait_gather(jnp.int32(NWIN - 1), row0, gsem0)
    pltpu.sync_copy(row0, acc.at[dst_all.at[NWIN - 1]], add=True)

    plsc.subcore_barrier()

    pltpu.sync_copy(acc.at[pl.ds(_al8(rbase), TR)],
                    out_hbm.at[cid, pl.ds(_al8(rbase), TR)])

    @pl.when(sid == NS - 1)
    def _():
        pltpu.sync_copy(acc.at[pl.ds(TAILB, 16)], out_hbm.at[cid, pl.ds(TAILB, 16)])


R = 1000
GR = N // R


def _t0_body(degp, x, w0, dinv_o, zs_o):
    d = 1.0 + degp[0, :, 0:1] + degp[1, :, 0:1]
    dinv = lax.rsqrt(d)
    dinv_o[...] = dinv
    zs_o[...] = jnp.dot(x[...], w0[...], preferred_element_type=jnp.float32) * dinv


def _t0(degp, x, w0):
    return pl.pallas_call(
        _t0_body,
        grid=(GR,),
        in_specs=[
            pl.BlockSpec((2, R, F), lambda i: (0, i, 0)),
            pl.BlockSpec((R, F), lambda i: (i, 0)),
            pl.BlockSpec((F, F), lambda i: (0, 0)),
        ],
        out_specs=[
            pl.BlockSpec((R, 1), lambda i: (i, 0)),
            pl.BlockSpec((R, F), lambda i: (i, 0)),
        ],
        out_shape=[
            jax.ShapeDtypeStruct((N, 1), jnp.float32),
            jax.ShapeDtypeStruct((N, F), jnp.float32),
        ],
    )(degp, x, w0)


def _t1_body(p, zs0, b0, dinv, w1, c0_o, zs1_o):
    c0 = dinv[...] * (p[0] + p[1] + zs0[...]) + b0[...]
    c0_o[...] = c0
    m = jnp.maximum(c0, 0.0)
    zs1_o[...] = jnp.dot(m, w1[...], preferred_element_type=jnp.float32) * dinv[...]


def _t1(p, zs0, b0, dinv, w1):
    return pl.pallas_call(
        _t1_body,
        grid=(GR,),
        in_specs=[
            pl.BlockSpec((2, R, F), lambda i: (0, i, 0)),
            pl.BlockSpec((R, F), lambda i: (i, 0)),
            pl.BlockSpec((1, F), lambda i: (0, 0)),
            pl.BlockSpec((R, 1), lambda i: (i, 0)),
            pl.BlockSpec((F, F), lambda i: (0, 0)),
        ],
        out_specs=[
            pl.BlockSpec((R, F), lambda i: (i, 0)),
            pl.BlockSpec((R, F), lambda i: (i, 0)),
        ],
        out_shape=[
            jax.ShapeDtypeStruct((N, F), jnp.float32),
            jax.ShapeDtypeStruct((N, F), jnp.float32),
        ],
    )(p, zs0, b0, dinv, w1)


def _tg(pp, zs, b, dinv, c0, g, bt, awh, awx, ab, w):
    def body(pp_r, zs_r, b_r, dinv_r, c0_r, g_r, bt_r, awh_r, awx_r, ab_r, w_r,
             zs_o, st):
        ph = pl.program_id(0)
        i = pl.program_id(1)
        c = dinv_r[...] * (pp_r[0] + pp_r[1] + zs_r[...]) + b_r[...]

        @pl.when(ph == 0)
        def _():
            s = jnp.sum(c, axis=0, keepdims=True)
            sq = jnp.sum(c * c, axis=0, keepdims=True)

            @pl.when(i == 0)
            def _():
                st[0:1] = s
                st[1:2] = sq

            @pl.when(i > 0)
            def _():
                st[0:1] += s
                st[1:2] += sq

        @pl.when(ph == 1)
        def _():
            mu = st[0:1] * (1.0 / N)
            var = st[1:2] * (1.0 / N) - mu * mu
            hn = (c - mu) * lax.rsqrt(var + 1e-5) * g_r[...] + bt_r[...]
            logit = (jnp.dot(hn, awh_r[...], preferred_element_type=jnp.float32)
                     + jnp.dot(c0_r[...], awx_r[...], preferred_element_type=jnp.float32)
                     + ab_r[...])
            alpha = 1.0 / (1.0 + jnp.exp(-logit))
            m = jnp.maximum((1.0 - alpha) * hn + alpha * c0_r[...], 0.0)
            zs_o[...] = jnp.dot(m, w_r[...],
                                preferred_element_type=jnp.float32) * dinv_r[...]

    return pl.pallas_call(
        body,
        grid=(2, GR),
        in_specs=[
            pl.BlockSpec((2, R, F), lambda p, i: (0, i, 0)),
            pl.BlockSpec((R, F), lambda p, i: (i, 0)),
            pl.BlockSpec((1, F), lambda p, i: (0, 0)),
            pl.BlockSpec((R, 1), lambda p, i: (i, 0)),
            pl.BlockSpec((R, F), lambda p, i: (i, 0)),
            pl.BlockSpec((1, F), lambda p, i: (0, 0)),
            pl.BlockSpec((1, F), lambda p, i: (0, 0)),
            pl.BlockSpec((F, 1), lambda p, i: (0, 0)),
            pl.BlockSpec((F, 1), lambda p, i: (0, 0)),
            pl.BlockSpec((1, 1), lambda p, i: (0, 0)),
            pl.BlockSpec((F, F), lambda p, i: (0, 0)),
        ],
        out_specs=pl.BlockSpec((R, F), lambda p, i: (i, 0)),
        out_shape=jax.ShapeDtypeStruct((N, F), jnp.float32),
        scratch_shapes=[pltpu.VMEM((2, F), jnp.float32)],
    )(pp, zs, b, dinv, c0, g, bt, awh, awx, ab, w)


def _tfin(pp, zs, b, dinv, g, bt):
    def body(pp_r, zs_r, b_r, dinv_r, g_r, bt_r, o, st):
        ph = pl.program_id(0)
        i = pl.program_id(1)
        c = dinv_r[...] * (pp_r[0] + pp_r[1] + zs_r[...]) + b_r[...]

        @pl.when(ph == 0)
        def _():
            s = jnp.sum(c, axis=0, keepdims=True)
            sq = jnp.sum(c * c, axis=0, keepdims=True)

            @pl.when(i == 0)
            def _():
                st[0:1] = s
                st[1:2] = sq

            @pl.when(i > 0)
            def _():
                st[0:1] += s
                st[1:2] += sq

        @pl.when(ph == 1)
        def _():
            mu = st[0:1] * (1.0 / N)
            var = st[1:2] * (1.0 / N) - mu * mu
            o[...] = (c - mu) * lax.rsqrt(var + 1e-5) * g_r[...] + bt_r[...]

    return pl.pallas_call(
        body,
        grid=(2, GR),
        in_specs=[
            pl.BlockSpec((2, R, F), lambda p, i: (0, i, 0)),
            pl.BlockSpec((R, F), lambda p, i: (i, 0)),
            pl.BlockSpec((1, F), lambda p, i: (0, 0)),
            pl.BlockSpec((R, 1), lambda p, i: (i, 0)),
            pl.BlockSpec((1, F), lambda p, i: (0, 0)),
            pl.BlockSpec((1, F), lambda p, i: (0, 0)),
        ],
        out_specs=pl.BlockSpec((R, F), lambda p, i: (i, 0)),
        out_shape=jax.ShapeDtypeStruct((N, F), jnp.float32),
        scratch_shapes=[pltpu.VMEM((2, F), jnp.float32)],
    )(pp, zs, b, dinv, g, bt)


def kernel(x, edge_index, Ws, bs, gammas, betas, attW, attb):
    src = edge_index[0].astype(jnp.int32).reshape(NW, 1, EPT)
    dst = edge_index[1].astype(jnp.int32).reshape(NW, NWIN, WIN)

    degp = _sc_deg(dst)
    dinv, zs = _t0(degp, x, Ws[0])

    p = _sc_conv(zs, src, dst)
    c0, zs = _t1(p, zs, bs[0][None], dinv, Ws[1])

    awh = attW[:F]
    awx = attW[F:]
    ab = attb.reshape(1, 1)

    for i in (1, 2):
        p = _sc_conv(zs, src, dst)
        zs = _tg(p, zs, bs[i][None], dinv, c0, gammas[i - 1][None],
                 betas[i - 1][None], awh, awx, ab, Ws[i + 1])

    p = _sc_conv(zs, src, dst)
    return _tfin(p, zs, bs[3][None], dinv, gammas[2][None], betas[2][None])


# speedup vs baseline: 1.0291x; 1.0291x over previous
import functools

import jax
import jax.numpy as jnp
from jax import lax
from jax.experimental import pallas as pl
from jax.experimental.pallas import tpu as pltpu
from jax.experimental.pallas import tpu_sc as plsc

N = 10000
F = 128
E = 320000
NC = 2
NS = 16
NW = NC * NS
EPT = E // NW
WIN = 80
NWIN = EPT // WIN
TR = 624
SR = 208
TAILB = NS * TR
DW = 16

_mesh = plsc.VectorSubcoreMesh(core_axis_name="c", subcore_axis_name="s")


def _al8(i):
    return pl.multiple_of(i, 8)


@functools.partial(
    pl.kernel,
    out_type=jax.ShapeDtypeStruct((NC, N, F), jnp.float32),
    mesh=_mesh,
    scratch_types=[
        pltpu.VMEM_SHARED((N, F), jnp.float32),
        pltpu.VMEM((NWIN, WIN), jnp.int32),
        pltpu.VMEM((WIN, F), jnp.float32),
    ],
)
def _sc_deg(dst_hbm, out_hbm, dacc, dst_all, obuf):
    cid = lax.axis_index("c")
    sid = lax.axis_index("s")
    wid = cid * NS + sid
    rbase = sid * TR

    @pl.loop(0, WIN)
    def _(i):
        @pl.loop(0, F, step=16)
        def _(j):
            obuf[i, pl.ds(j, 16)] = jnp.zeros((16,), jnp.float32)

    @pl.loop(0, 7 * WIN, step=WIN)
    def _(r):
        pltpu.sync_copy(obuf, dacc.at[pl.ds(_al8(rbase + r), WIN)])

    pltpu.sync_copy(obuf.at[pl.ds(0, 64)], dacc.at[pl.ds(_al8(rbase + 7 * WIN), 64)])

    @pl.when(sid == NS - 1)
    def _():
        pltpu.sync_copy(obuf.at[pl.ds(0, 16)], dacc.at[pl.ds(TAILB, 16)])

    @pl.loop(0, WIN)
    def _(i):
        @pl.loop(0, F, step=16)
        def _(j):
            obuf[i, pl.ds(j, 16)] = jnp.ones((16,), jnp.float32)

    pltpu.sync_copy(dst_hbm.at[wid], dst_all)
    plsc.subcore_barrier()

    @pl.loop(0, NWIN)
    def _(w):
        pltpu.sync_copy(obuf, dacc.at[dst_all.at[w]], add=True)

    plsc.subcore_barrier()

    pltpu.sync_copy(dacc.at[pl.ds(_al8(rbase), TR)],
                    out_hbm.at[cid, pl.ds(_al8(rbase), TR)])

    @pl.when(sid == NS - 1)
    def _():
        pltpu.sync_copy(dacc.at[pl.ds(TAILB, 16)], out_hbm.at[cid, pl.ds(TAILB, 16)])


@functools.partial(
    pl.kernel,
    out_type=jax.ShapeDtypeStruct((NC, N, F), jnp.float32),
    mesh=_mesh,
    scratch_types=[
        pltpu.VMEM_SHARED((N, F), jnp.float32),
        pltpu.VMEM((EPT,), jnp.int32),
        pltpu.VMEM((NWIN, WIN), jnp.int32),
        pltpu.VMEM((WIN, F), jnp.float32),
        pltpu.VMEM((WIN, F), jnp.float32),
        pltpu.SemaphoreType.DMA,
        pltpu.SemaphoreType.DMA,
    ],
)
def _sc_conv(zs_hbm, src_hbm, dst_hbm, out_hbm,
             acc, src_all, dst_all, row0, row1, gsem0, gsem1):
    cid = lax.axis_index("c")
    sid = lax.axis_index("s")
    wid = cid * NS + sid
    rbase = sid * TR

    @pl.loop(0, WIN)
    def _(i):
        @pl.loop(0, F, step=16)
        def _(j):
            row0[i, pl.ds(j, 16)] = jnp.zeros((16,), jnp.float32)

    @pl.loop(0, 7 * WIN, step=WIN)
    def _(r):
        pltpu.sync_copy(row0, acc.at[pl.ds(_al8(rbase + r), WIN)])

    pltpu.sync_copy(row0.at[pl.ds(0, 64)], acc.at[pl.ds(_al8(rbase + 7 * WIN), 64)])

    @pl.when(sid == NS - 1)
    def _():
        pltpu.sync_copy(row0.at[pl.ds(0, 16)], acc.at[pl.ds(TAILB, 16)])

    pltpu.sync_copy(src_hbm.at[wid, 0], src_all)
    pltpu.sync_copy(dst_hbm.at[wid], dst_all)
    plsc.subcore_barrier()

    def start_gather(w, row, sem):
        pltpu.async_copy(zs_hbm.at[src_all.at[pl.ds(_al8(w * WIN), WIN)]], row, sem)

    def wait_gather(w, row, sem):
        pltpu.make_async_copy(
            zs_hbm.at[src_all.at[pl.ds(_al8(w * WIN), WIN)]], row, sem).wait()

    start_gather(jnp.int32(0), row0, gsem0)
    start_gather(jnp.int32(1), row1, gsem1)

    @pl.loop(0, NWIN - 1, step=2)
    def _(k):
        for b, (row, sem) in enumerate(((row0, gsem0), (row1, gsem1))):
            w = k + b
            wait_gather(w, row, sem)
            pltpu.sync_copy(row, acc.at[dst_all.at[w]], add=True)
            nxt = w + 2

            @pl.when(nxt < NWIN)
            def _():
                start_gather(nxt, row, sem)

    wait_gather(jnp.int32(NWIN - 1), row0, gsem0)
    pltpu.sync_copy(row0, acc.at[dst_all.at[NWIN - 1]], add=True)

    plsc.subcore_barrier()

    pltpu.sync_copy(acc.at[pl.ds(_al8(rbase), TR)],
                    out_hbm.at[cid, pl.ds(_al8(rbase), TR)])

    @pl.when(sid == NS - 1)
    def _():
        pltpu.sync_copy(acc.at[pl.ds(TAILB, 16)], out_hbm.at[cid, pl.ds(TAILB, 16)])


R = 1000
GR = N // R


def _t0_body(degp, x, w0, dinv_o, zs_o):
    d = 1.0 + degp[0, :, 0:1] + degp[1, :, 0:1]
    dinv = lax.rsqrt(d)
    dinv_o[...] = dinv
    zs_o[...] = jnp.dot(x[...], w0[...], preferred_element_type=jnp.float32) * dinv


def _t0(degp, x, w0):
    return pl.pallas_call(
        _t0_body,
        grid=(GR,),
        in_specs=[
            pl.BlockSpec((2, R, F), lambda i: (0, i, 0)),
            pl.BlockSpec((R, F), lambda i: (i, 0)),
            pl.BlockSpec((F, F), lambda i: (0, 0)),
        ],
        out_specs=[
            pl.BlockSpec((R, 1), lambda i: (i, 0)),
            pl.BlockSpec((R, F), lambda i: (i, 0)),
        ],
        out_shape=[
            jax.ShapeDtypeStruct((N, 1), jnp.float32),
            jax.ShapeDtypeStruct((N, F), jnp.float32),
        ],
    )(degp, x, w0)


def _t1_body(p, zs0, b0, dinv, w1, c0_o, zs1_o):
    c0 = dinv[...] * (p[0] + p[1] + zs0[...]) + b0[...]
    c0_o[...] = c0
    m = jnp.maximum(c0, 0.0)
    zs1_o[...] = jnp.dot(m, w1[...], preferred_element_type=jnp.float32) * dinv[...]


def _t1(p, zs0, b0, dinv, w1):
    return pl.pallas_call(
        _t1_body,
        grid=(GR,),
        in_specs=[
            pl.BlockSpec((2, R, F), lambda i: (0, i, 0)),
            pl.BlockSpec((R, F), lambda i: (i, 0)),
            pl.BlockSpec((1, F), lambda i: (0, 0)),
            pl.BlockSpec((R, 1), lambda i: (i, 0)),
            pl.BlockSpec((F, F), lambda i: (0, 0)),
        ],
        out_specs=[
            pl.BlockSpec((R, F), lambda i: (i, 0)),
            pl.BlockSpec((R, F), lambda i: (i, 0)),
        ],
        out_shape=[
            jax.ShapeDtypeStruct((N, F), jnp.float32),
            jax.ShapeDtypeStruct((N, F), jnp.float32),
        ],
    )(p, zs0, b0, dinv, w1)


def _ta_body(p, zs, b, dinv, c_o, st_o):
    i = pl.program_id(0)
    c = dinv[...] * (p[0] + p[1] + zs[...]) + b[...]
    c_o[...] = c
    s = jnp.sum(c, axis=0, keepdims=True)
    sq = jnp.sum(c * c, axis=0, keepdims=True)
    st = jnp.concatenate([s, sq], axis=0)

    @pl.when(i == 0)
    def _():
        st_o[...] = st

    @pl.when(i > 0)
    def _():
        st_o[...] += st


def _ta(p, zs, b, dinv):
    return pl.pallas_call(
        _ta_body,
        grid=(GR,),
        in_specs=[
            pl.BlockSpec((2, R, F), lambda i: (0, i, 0)),
            pl.BlockSpec((R, F), lambda i: (i, 0)),
            pl.BlockSpec((1, F), lambda i: (0, 0)),
            pl.BlockSpec((R, 1), lambda i: (i, 0)),
        ],
        out_specs=[
            pl.BlockSpec((R, F), lambda i: (i, 0)),
            pl.BlockSpec((2, F), lambda i: (0, 0)),
        ],
        out_shape=[
            jax.ShapeDtypeStruct((N, F), jnp.float32),
            jax.ShapeDtypeStruct((2, F), jnp.float32),
        ],
    )(p, zs, b, dinv)


def _tb_body(c, st, c0, dinv, g, bt, awh, awx, ab, w, zs_o):
    mu = st[0:1] * (1.0 / N)
    var = st[1:2] * (1.0 / N) - mu * mu
    hn = (c[...] - mu) * lax.rsqrt(var + 1e-5) * g[...] + bt[...]
    logit = (jnp.dot(hn, awh[...], preferred_element_type=jnp.float32)
             + jnp.dot(c0[...], awx[...], preferred_element_type=jnp.float32)
             + ab[...])
    alpha = 1.0 / (1.0 + jnp.exp(-logit))
    m = jnp.maximum((1.0 - alpha) * hn + alpha * c0[...], 0.0)
    zs_o[...] = jnp.dot(m, w[...], preferred_element_type=jnp.float32) * dinv[...]


def _tb(c, st, c0, dinv, g, bt, awh, awx, ab, w):
    return pl.pallas_call(
        _tb_body,
        grid=(GR,),
        in_specs=[
            pl.BlockSpec((R, F), lambda i: (i, 0)),
            pl.BlockSpec((2, F), lambda i: (0, 0)),
            pl.BlockSpec((R, F), lambda i: (i, 0)),
            pl.BlockSpec((R, 1), lambda i: (i, 0)),
            pl.BlockSpec((1, F), lambda i: (0, 0)),
            pl.BlockSpec((1, F), lambda i: (0, 0)),
            pl.BlockSpec((F, 1), lambda i: (0, 0)),
            pl.BlockSpec((F, 1), lambda i: (0, 0)),
            pl.BlockSpec((1, 1), lambda i: (0, 0)),
            pl.BlockSpec((F, F), lambda i: (0, 0)),
        ],
        out_specs=pl.BlockSpec((R, F), lambda i: (i, 0)),
        out_shape=jax.ShapeDtypeStruct((N, F), jnp.float32),
    )(c, st, c0, dinv, g, bt, awh, awx, ab, w)


def _tf_body(c, st, g, bt, o):
    mu = st[0:1] * (1.0 / N)
    var = st[1:2] * (1.0 / N) - mu * mu
    o[...] = (c[...] - mu) * lax.rsqrt(var + 1e-5) * g[...] + bt[...]


def _tf(c, st, g, bt):
    return pl.pallas_call(
        _tf_body,
        grid=(GR,),
        in_specs=[
            pl.BlockSpec((R, F), lambda i: (i, 0)),
            pl.BlockSpec((2, F), lambda i: (0, 0)),
            pl.BlockSpec((1, F), lambda i: (0, 0)),
            pl.BlockSpec((1, F), lambda i: (0, 0)),
        ],
        out_specs=pl.BlockSpec((R, F), lambda i: (i, 0)),
        out_shape=jax.ShapeDtypeStruct((N, F), jnp.float32),
    )(c, st, g, bt)


def kernel(x, edge_index, Ws, bs, gammas, betas, attW, attb):
    src = edge_index[0].astype(jnp.int32).reshape(NW, 1, EPT)
    dst = edge_index[1].astype(jnp.int32).reshape(NW, NWIN, WIN)

    degp = _sc_deg(dst)
    dinv, zs = _t0(degp, x, Ws[0])

    p = _sc_conv(zs, src, dst)
    c0, zs = _t1(p, zs, bs[0][None], dinv, Ws[1])

    awh = attW[:F]
    awx = attW[F:]
    ab = attb.reshape(1, 1)

    c = st = None
    for i in (1, 2, 3):
        p = _sc_conv(zs, src, dst)
        c, st = _ta(p, zs, bs[i][None], dinv)
        if i < 3:
            zs = _tb(c, st, c0, dinv, gammas[i - 1][None], betas[i - 1][None],
                     awh, awx, ab, Ws[i + 1])

    return _tf(c, st, gammas[2][None], betas[2][None])


# speedup vs baseline: 1.0542x; 1.0244x over previous
import functools

import jax
import jax.numpy as jnp
from jax import lax
from jax.experimental import pallas as pl
from jax.experimental.pallas import tpu as pltpu
from jax.experimental.pallas import tpu_sc as plsc

N = 10000
F = 128
E = 320000
NC = 2
NS = 16
NW = NC * NS
EPT = E // NW
WIN = 80
NWIN = EPT // WIN
TR = 624
SR = 208
TAILB = NS * TR
DW = 16

_mesh = plsc.VectorSubcoreMesh(core_axis_name="c", subcore_axis_name="s")


def _al8(i):
    return pl.multiple_of(i, 8)


@functools.partial(
    pl.kernel,
    out_type=jax.ShapeDtypeStruct((NC, N, F), jnp.float32),
    mesh=_mesh,
    scratch_types=[
        pltpu.VMEM_SHARED((N, F), jnp.float32),
        pltpu.VMEM((NWIN, WIN), jnp.int32),
        pltpu.VMEM((WIN, F), jnp.float32),
    ],
)
def _sc_deg(dst_hbm, out_hbm, dacc, dst_all, obuf):
    cid = lax.axis_index("c")
    sid = lax.axis_index("s")
    wid = cid * NS + sid
    rbase = sid * TR

    @pl.loop(0, WIN)
    def _(i):
        @pl.loop(0, F, step=16)
        def _(j):
            obuf[i, pl.ds(j, 16)] = jnp.zeros((16,), jnp.float32)

    @pl.loop(0, 7 * WIN, step=WIN)
    def _(r):
        pltpu.sync_copy(obuf, dacc.at[pl.ds(_al8(rbase + r), WIN)])

    pltpu.sync_copy(obuf.at[pl.ds(0, 64)], dacc.at[pl.ds(_al8(rbase + 7 * WIN), 64)])

    @pl.when(sid == NS - 1)
    def _():
        pltpu.sync_copy(obuf.at[pl.ds(0, 16)], dacc.at[pl.ds(TAILB, 16)])

    @pl.loop(0, WIN)
    def _(i):
        @pl.loop(0, F, step=16)
        def _(j):
            obuf[i, pl.ds(j, 16)] = jnp.ones((16,), jnp.float32)

    pltpu.sync_copy(dst_hbm.at[wid], dst_all)
    plsc.subcore_barrier()

    @pl.loop(0, NWIN)
    def _(w):
        pltpu.sync_copy(obuf, dacc.at[dst_all.at[w]], add=True)

    plsc.subcore_barrier()

    pltpu.sync_copy(dacc.at[pl.ds(_al8(rbase), TR)],
                    out_hbm.at[cid, pl.ds(_al8(rbase), TR)])

    @pl.when(sid == NS - 1)
    def _():
        pltpu.sync_copy(dacc.at[pl.ds(TAILB, 16)], out_hbm.at[cid, pl.ds(TAILB, 16)])


@functools.partial(
    pl.kernel,
    out_type=jax.ShapeDtypeStruct((NC, N, F), jnp.float32),
    mesh=_mesh,
    scratch_types=[
        pltpu.VMEM_SHARED((N, F), jnp.float32),
        pltpu.VMEM((EPT,), jnp.int32),
        pltpu.VMEM((NWIN, WIN), jnp.int32),
        pltpu.VMEM((WIN, F), jnp.float32),
        pltpu.VMEM((WIN, F), jnp.float32),
        pltpu.SemaphoreType.DMA,
        pltpu.SemaphoreType.DMA,
    ],
)
def _sc_conv(zs_hbm, src_hbm, dst_hbm, out_hbm,
             acc, src_all, dst_all, row0, row1, gsem0, gsem1):
    cid = lax.axis_index("c")
    sid = lax.axis_index("s")
    wid = cid * NS + sid
    rbase = sid * TR

    @pl.loop(0, WIN)
    def _(i):
        @pl.loop(0, F, step=16)
        def _(j):
            row0[i, pl.ds(j, 16)] = jnp.zeros((16,), jnp.float32)

    @pl.loop(0, 7 * WIN, step=WIN)
    def _(r):
        pltpu.sync_copy(row0, acc.at[pl.ds(_al8(rbase + r), WIN)])

    pltpu.sync_copy(row0.at[pl.ds(0, 64)], acc.at[pl.ds(_al8(rbase + 7 * WIN), 64)])

    @pl.when(sid == NS - 1)
    def _():
        pltpu.sync_copy(row0.at[pl.ds(0, 16)], acc.at[pl.ds(TAILB, 16)])

    pltpu.sync_copy(src_hbm.at[wid, 0], src_all)
    pltpu.sync_copy(dst_hbm.at[wid], dst_all)
    plsc.subcore_barrier()

    def start_gather(w, row, sem):
        pltpu.async_copy(zs_hbm.at[src_all.at[pl.ds(_al8(w * WIN), WIN)]], row, sem)

    def wait_gather(w, row, sem):
        pltpu.make_async_copy(
            zs_hbm.at[src_all.at[pl.ds(_al8(w * WIN), WIN)]], row, sem).wait()

    start_gather(jnp.int32(0), row0, gsem0)
    start_gather(jnp.int32(1), row1, gsem1)

    @pl.loop(0, NWIN - 1, step=2)
    def _(k):
        for b, (row, sem) in enumerate(((row0, gsem0), (row1, gsem1))):
            w = k + b
            wait_gather(w, row, sem)
            pltpu.sync_copy(row, acc.at[dst_all.at[w]], add=True)
            nxt = w + 2

            @pl.when(nxt < NWIN)
            def _():
                start_gather(nxt, row, sem)

    wait_gather(jnp.int32(NWIN - 1), row0, gsem0)
    pltpu.sync_copy(row0, acc.at[dst_all.at[NWIN - 1]], add=True)

    plsc.subcore_barrier()

    pltpu.sync_copy(acc.at[pl.ds(_al8(rbase), TR)],
                    out_hbm.at[cid, pl.ds(_al8(rbase), TR)])

    @pl.when(sid == NS - 1)
    def _():
        pltpu.sync_copy(acc.at[pl.ds(TAILB, 16)], out_hbm.at[cid, pl.ds(TAILB, 16)])


R = 2000
GR = N // R


def _t0_body(degp, x, w0, dinv_o, zs_o):
    d = 1.0 + degp[0, :, 0:1] + degp[1, :, 0:1]
    dinv = lax.rsqrt(d)
    dinv_o[...] = dinv
    zs_o[...] = jnp.dot(x[...], w0[...], preferred_element_type=jnp.float32) * dinv


def _t0(degp, x, w0):
    return pl.pallas_call(
        _t0_body,
        grid=(GR,),
        in_specs=[
            pl.BlockSpec((2, R, F), lambda i: (0, i, 0)),
            pl.BlockSpec((R, F), lambda i: (i, 0)),
            pl.BlockSpec((F, F), lambda i: (0, 0)),
        ],
        out_specs=[
            pl.BlockSpec((R, 1), lambda i: (i, 0)),
            pl.BlockSpec((R, F), lambda i: (i, 0)),
        ],
        out_shape=[
            jax.ShapeDtypeStruct((N, 1), jnp.float32),
            jax.ShapeDtypeStruct((N, F), jnp.float32),
        ],
    )(degp, x, w0)


def _t1_body(p, zs0, b0, dinv, w1, c0_o, zs1_o):
    c0 = dinv[...] * (p[0] + p[1] + zs0[...]) + b0[...]
    c0_o[...] = c0
    m = jnp.maximum(c0, 0.0)
    zs1_o[...] = jnp.dot(m, w1[...], preferred_element_type=jnp.float32) * dinv[...]


def _t1(p, zs0, b0, dinv, w1):
    return pl.pallas_call(
        _t1_body,
        grid=(GR,),
        in_specs=[
            pl.BlockSpec((2, R, F), lambda i: (0, i, 0)),
            pl.BlockSpec((R, F), lambda i: (i, 0)),
            pl.BlockSpec((1, F), lambda i: (0, 0)),
            pl.BlockSpec((R, 1), lambda i: (i, 0)),
            pl.BlockSpec((F, F), lambda i: (0, 0)),
        ],
        out_specs=[
            pl.BlockSpec((R, F), lambda i: (i, 0)),
            pl.BlockSpec((R, F), lambda i: (i, 0)),
        ],
        out_shape=[
            jax.ShapeDtypeStruct((N, F), jnp.float32),
            jax.ShapeDtypeStruct((N, F), jnp.float32),
        ],
    )(p, zs0, b0, dinv, w1)


def _ta_body(p, zs, b, dinv, c_o, st_o):
    i = pl.program_id(0)
    c = dinv[...] * (p[0] + p[1] + zs[...]) + b[...]
    c_o[...] = c
    s = jnp.sum(c, axis=0, keepdims=True)
    sq = jnp.sum(c * c, axis=0, keepdims=True)
    st = jnp.concatenate([s, sq], axis=0)

    @pl.when(i == 0)
    def _():
        st_o[...] = st

    @pl.when(i > 0)
    def _():
        st_o[...] += st


def _ta(p, zs, b, dinv):
    return pl.pallas_call(
        _ta_body,
        grid=(GR,),
        in_specs=[
            pl.BlockSpec((2, R, F), lambda i: (0, i, 0)),
            pl.BlockSpec((R, F), lambda i: (i, 0)),
            pl.BlockSpec((1, F), lambda i: (0, 0)),
            pl.BlockSpec((R, 1), lambda i: (i, 0)),
        ],
        out_specs=[
            pl.BlockSpec((R, F), lambda i: (i, 0)),
            pl.BlockSpec((2, F), lambda i: (0, 0)),
        ],
        out_shape=[
            jax.ShapeDtypeStruct((N, F), jnp.float32),
            jax.ShapeDtypeStruct((2, F), jnp.float32),
        ],
    )(p, zs, b, dinv)


def _tb_body(c, st, c0, dinv, g, bt, awh, awx, ab, w, zs_o):
    mu = st[0:1] * (1.0 / N)
    var = st[1:2] * (1.0 / N) - mu * mu
    hn = (c[...] - mu) * lax.rsqrt(var + 1e-5) * g[...] + bt[...]
    logit = (jnp.dot(hn, awh[...], preferred_element_type=jnp.float32)
             + jnp.dot(c0[...], awx[...], preferred_element_type=jnp.float32)
             + ab[...])
    alpha = 1.0 / (1.0 + jnp.exp(-logit))
    m = jnp.maximum((1.0 - alpha) * hn + alpha * c0[...], 0.0)
    zs_o[...] = jnp.dot(m, w[...], preferred_element_type=jnp.float32) * dinv[...]


def _tb(c, st, c0, dinv, g, bt, awh, awx, ab, w):
    return pl.pallas_call(
        _tb_body,
        grid=(GR,),
        in_specs=[
            pl.BlockSpec((R, F), lambda i: (i, 0)),
            pl.BlockSpec((2, F), lambda i: (0, 0)),
            pl.BlockSpec((R, F), lambda i: (i, 0)),
            pl.BlockSpec((R, 1), lambda i: (i, 0)),
            pl.BlockSpec((1, F), lambda i: (0, 0)),
            pl.BlockSpec((1, F), lambda i: (0, 0)),
            pl.BlockSpec((F, 1), lambda i: (0, 0)),
            pl.BlockSpec((F, 1), lambda i: (0, 0)),
            pl.BlockSpec((1, 1), lambda i: (0, 0)),
            pl.BlockSpec((F, F), lambda i: (0, 0)),
        ],
        out_specs=pl.BlockSpec((R, F), lambda i: (i, 0)),
        out_shape=jax.ShapeDtypeStruct((N, F), jnp.float32),
    )(c, st, c0, dinv, g, bt, awh, awx, ab, w)


def _tf_body(c, st, g, bt, o):
    mu = st[0:1] * (1.0 / N)
    var = st[1:2] * (1.0 / N) - mu * mu
    o[...] = (c[...] - mu) * lax.rsqrt(var + 1e-5) * g[...] + bt[...]


def _tf(c, st, g, bt):
    return pl.pallas_call(
        _tf_body,
        grid=(GR,),
        in_specs=[
            pl.BlockSpec((R, F), lambda i: (i, 0)),
            pl.BlockSpec((2, F), lambda i: (0, 0)),
            pl.BlockSpec((1, F), lambda i: (0, 0)),
            pl.BlockSpec((1, F), lambda i: (0, 0)),
        ],
        out_specs=pl.BlockSpec((R, F), lambda i: (i, 0)),
        out_shape=jax.ShapeDtypeStruct((N, F), jnp.float32),
    )(c, st, g, bt)


def kernel(x, edge_index, Ws, bs, gammas, betas, attW, attb):
    src = edge_index[0].astype(jnp.int32).reshape(NW, 1, EPT)
    dst = edge_index[1].astype(jnp.int32).reshape(NW, NWIN, WIN)

    degp = _sc_deg(dst)
    dinv, zs = _t0(degp, x, Ws[0])

    p = _sc_conv(zs, src, dst)
    c0, zs = _t1(p, zs, bs[0][None], dinv, Ws[1])

    awh = attW[:F]
    awx = attW[F:]
    ab = attb.reshape(1, 1)

    c = st = None
    for i in (1, 2, 3):
        p = _sc_conv(zs, src, dst)
        c, st = _ta(p, zs, bs[i][None], dinv)
        if i < 3:
            zs = _tb(c, st, c0, dinv, gammas[i - 1][None], betas[i - 1][None],
                     awh, awx, ab, Ws[i + 1])

    return _tf(c, st, gammas[2][None], betas[2][None])
